# Initial kernel scaffold; baseline (speedup 1.0000x reference)
import jax
import jax.numpy as jnp
from jax.experimental import pallas as pl


def kernel(cdfg_xs, cdfg_as, W_in, b_in, W0, b0, W1, b1, W2, b2, graph, coverpoint, coverpoint_mask):
    raise NotImplementedError("write your pallas kernel here")


# speedup vs baseline: 2.3553x; 2.3553x over previous
import functools

import jax
import jax.numpy as jnp
from jax.experimental import pallas as pl
from jax.experimental.pallas import tpu as pltpu

G, N, F, H, B = 8, 1024, 128, 64, 16


def _gnn_body(xs_ref, a_ref, win_ref, bin_ref, w0_ref, b0_ref, w1_ref,
              b1_ref, w2_ref, b2_ref, out_ref):
    a = a_ref[0]
    x = jnp.maximum(
        jnp.dot(xs_ref[0], win_ref[...], preferred_element_type=jnp.float32)
        + bin_ref[...], 0.0)
    to_add = x
    x = jnp.maximum(
        jnp.dot(a, jnp.dot(x, w0_ref[...], preferred_element_type=jnp.float32),
                preferred_element_type=jnp.float32) + b0_ref[...], 0.0)
    x = jnp.maximum(
        jnp.dot(a, jnp.dot(x, w1_ref[...], preferred_element_type=jnp.float32),
                preferred_element_type=jnp.float32) + b1_ref[...], 0.0)
    y = jnp.dot(a, jnp.dot(x, w2_ref[...], preferred_element_type=jnp.float32),
                preferred_element_type=jnp.float32) + b2_ref[...]
    y = y - jnp.max(y, axis=-1, keepdims=True)
    e = jnp.exp(y)
    x = e / jnp.sum(e, axis=-1, keepdims=True)
    out_ref[0] = x + to_add


def _pool_body(gids_ref, hid_ref, mask_ref, out_ref):
    del gids_ref
    m = mask_ref[0, 0]
    x = hid_ref[0]
    s = jnp.sum(x * m[:, None], axis=0)
    c = jnp.sum(m)
    out_ref[0, 0] = s / jnp.maximum(c, 1.0)


@jax.jit
def kernel(cdfg_xs, cdfg_as, W_in, b_in, W0, b0, W1, b1, W2, b2, graph,
           coverpoint, coverpoint_mask):
    del coverpoint
    hidden = pl.pallas_call(
        _gnn_body,
        grid=(G,),
        in_specs=[
            pl.BlockSpec((1, N, F), lambda g: (g, 0, 0)),
            pl.BlockSpec((1, N, N), lambda g: (g, 0, 0)),
            pl.BlockSpec((F, H), lambda g: (0, 0)),
            pl.BlockSpec((1, H), lambda g: (0, 0)),
            pl.BlockSpec((H, H), lambda g: (0, 0)),
            pl.BlockSpec((1, H), lambda g: (0, 0)),
            pl.BlockSpec((H, H), lambda g: (0, 0)),
            pl.BlockSpec((1, H), lambda g: (0, 0)),
            pl.BlockSpec((H, H), lambda g: (0, 0)),
            pl.BlockSpec((1, H), lambda g: (0, 0)),
        ],
        out_specs=pl.BlockSpec((1, N, H), lambda g: (g, 0, 0)),
        out_shape=jax.ShapeDtypeStruct((G, N, H), jnp.float32),
    )(cdfg_xs, cdfg_as, W_in, b_in.reshape(1, H), W0, b0.reshape(1, H),
      W1, b1.reshape(1, H), W2, b2.reshape(1, H))

    gids = graph[:, 0].astype(jnp.int32)
    maskf = coverpoint_mask.astype(jnp.float32).reshape(B, 1, N)

    out = pl.pallas_call(
        _pool_body,
        grid_spec=pltpu.PrefetchScalarGridSpec(
            num_scalar_prefetch=1,
            grid=(B,),
            in_specs=[
                pl.BlockSpec((1, N, H), lambda b, gids: (gids[b], 0, 0)),
                pl.BlockSpec((1, 1, N), lambda b, gids: (b, 0, 0)),
            ],
            out_specs=pl.BlockSpec((1, 1, H), lambda b, gids: (b, 0, 0)),
        ),
        out_shape=jax.ShapeDtypeStruct((B, 1, H), jnp.float32),
    )(gids, hidden, maskf)
    return out.reshape(B, H)


# speedup vs baseline: 2.9661x; 1.2593x over previous
import jax
import jax.numpy as jnp
from jax.experimental import pallas as pl

G, N, F, H, B = 8, 1024, 128, 64, 16


def _gnn_body(xs_ref, a_ref, win_ref, bin_ref, w0_ref, b0_ref, w1_ref,
              b1_ref, w2_ref, b2_ref, gids_ref, mask_ref, out_ref):
    g = pl.program_id(0)
    a = a_ref[0]
    x = jnp.maximum(
        jnp.dot(xs_ref[0], win_ref[...], preferred_element_type=jnp.float32)
        + bin_ref[...], 0.0)
    to_add = x
    x = jnp.maximum(
        jnp.dot(a, jnp.dot(x, w0_ref[...], preferred_element_type=jnp.float32),
                preferred_element_type=jnp.float32) + b0_ref[...], 0.0)
    x = jnp.maximum(
        jnp.dot(a, jnp.dot(x, w1_ref[...], preferred_element_type=jnp.float32),
                preferred_element_type=jnp.float32) + b1_ref[...], 0.0)
    y = jnp.dot(a, jnp.dot(x, w2_ref[...], preferred_element_type=jnp.float32),
                preferred_element_type=jnp.float32) + b2_ref[...]
    y = y - jnp.max(y, axis=-1, keepdims=True)
    e = jnp.exp(y)
    x = e / jnp.sum(e, axis=-1, keepdims=True)
    x = x + to_add

    m = mask_ref[...]
    pm = jnp.dot(m, x, preferred_element_type=jnp.float32)
    cnt = jnp.maximum(jnp.sum(m, axis=1, keepdims=True), 1.0)
    pooled = pm / cnt
    sel = gids_ref[...] == g

    @pl.when(g == 0)
    def _init():
        out_ref[...] = jnp.zeros_like(out_ref)

    out_ref[...] = jnp.where(sel, pooled, out_ref[...])


@jax.jit
def kernel(cdfg_xs, cdfg_as, W_in, b_in, W0, b0, W1, b1, W2, b2, graph,
           coverpoint, coverpoint_mask):
    del coverpoint
    gids = graph.astype(jnp.int32).reshape(B, 1)
    maskf = coverpoint_mask.astype(jnp.float32)

    out = pl.pallas_call(
        _gnn_body,
        grid=(G,),
        in_specs=[
            pl.BlockSpec((1, N, F), lambda g: (g, 0, 0)),
            pl.BlockSpec((1, N, N), lambda g: (g, 0, 0)),
            pl.BlockSpec((F, H), lambda g: (0, 0)),
            pl.BlockSpec((1, H), lambda g: (0, 0)),
            pl.BlockSpec((H, H), lambda g: (0, 0)),
            pl.BlockSpec((1, H), lambda g: (0, 0)),
            pl.BlockSpec((H, H), lambda g: (0, 0)),
            pl.BlockSpec((1, H), lambda g: (0, 0)),
            pl.BlockSpec((H, H), lambda g: (0, 0)),
            pl.BlockSpec((1, H), lambda g: (0, 0)),
            pl.BlockSpec((B, 1), lambda g: (0, 0)),
            pl.BlockSpec((B, N), lambda g: (0, 0)),
        ],
        out_specs=pl.BlockSpec((B, H), lambda g: (0, 0)),
        out_shape=jax.ShapeDtypeStruct((B, H), jnp.float32),
    )(cdfg_xs, cdfg_as, W_in, b_in.reshape(1, H), W0, b0.reshape(1, H),
      W1, b1.reshape(1, H), W2, b2.reshape(1, H), gids, maskf)
    return out
